# baseline (speedup 1.0000x reference)
import jax
import jax.numpy as jnp
from jax.experimental import pallas as pl
from jax.experimental.pallas import tpu as pltpu

_TILE = 2048


def _moe_body(x_ref, gates_ref, bg_ref, we_ref, wg_ref, be_ref, p_ref, s_ref,
              out_ref, wall_ref):
    E, D, O = we_ref.shape
    EO = E * O

    @pl.when(pl.program_id(0) == 0)
    def _prep():
        for e in range(E):
            wall_ref[:, e * O : (e + 1) * O] = we_ref[e].astype(jnp.bfloat16)
        wall_ref[:, EO : EO + E] = wg_ref[...].astype(jnp.bfloat16)

    xb = x_ref[...].astype(jnp.bfloat16)
    y_all = jnp.dot(xb, wall_ref[...], preferred_element_type=jnp.float32)
    y = y_all[:, :EO]
    g = (y_all[:, EO : EO + E] + bg_ref[...]) * gates_ref[...]
    ge = jnp.dot(g.astype(jnp.bfloat16), p_ref[...],
                 preferred_element_type=jnp.float32)
    z = (ge * y).astype(jnp.bfloat16)
    out = jnp.dot(z, s_ref[...], preferred_element_type=jnp.float32)
    out_ref[...] = out + jnp.dot(g, be_ref[...], preferred_element_type=jnp.float32)


def kernel(x, gates, W_gate, b_gate, W_experts, b_experts):
    B, D = x.shape
    E = gates.shape[1]
    O = W_experts.shape[2]
    bg2 = b_gate.reshape(1, E)
    p_mat = jnp.repeat(jnp.eye(E, dtype=jnp.bfloat16), O, axis=1)
    s_mat = jnp.tile(jnp.eye(O, dtype=jnp.bfloat16), (E, 1))
    tile = _TILE if B % _TILE == 0 else B
    grid = (B // tile,)
    return pl.pallas_call(
        _moe_body,
        grid=grid,
        in_specs=[
            pl.BlockSpec((tile, D), lambda i: (i, 0)),
            pl.BlockSpec((tile, E), lambda i: (i, 0)),
            pl.BlockSpec((1, E), lambda i: (0, 0)),
            pl.BlockSpec((E, D, O), lambda i: (0, 0, 0)),
            pl.BlockSpec((D, E), lambda i: (0, 0)),
            pl.BlockSpec((E, O), lambda i: (0, 0)),
            pl.BlockSpec((E, E * O), lambda i: (0, 0)),
            pl.BlockSpec((E * O, O), lambda i: (0, 0)),
        ],
        out_specs=pl.BlockSpec((tile, O), lambda i: (i, 0)),
        out_shape=jax.ShapeDtypeStruct((B, O), jnp.float32),
        scratch_shapes=[pltpu.VMEM((D, E * O + E), jnp.bfloat16)],
    )(x, gates, bg2, W_experts, W_gate, b_experts, p_mat, s_mat)
